# baseline (speedup 1.0000x reference)
import jax
import jax.numpy as jnp
from jax.experimental import pallas as pl
from jax.experimental.pallas import tpu as pltpu

_MEDNUM = 2048
_FDIM = 64
_CH = 256
_NCHUNK = 2 * _MEDNUM // _CH
_NBUF = 4


def _ddi_gcn_kernel(adj_hbm, w_ref, inter_ref, out_ref, buf_ref, sems):
    w = w_ref[:]
    t = inter_ref[0, 0]

    def copy(i, slot):
        return pltpu.make_async_copy(
            adj_hbm.at[pl.ds(i * _CH, _CH), :],
            buf_ref.at[slot],
            sems.at[slot],
        )

    for s in range(_NBUF):
        copy(s, s).start()

    half = _MEDNUM // _CH
    for i in range(_NCHUNK):
        slot = i % _NBUF
        copy(i, slot).wait()
        a = buf_ref[slot, :, :_MEDNUM] + buf_ref[slot, :, _MEDNUM:]
        y = jnp.maximum(jnp.dot(a, w, preferred_element_type=jnp.float32), 0.0)
        if i < half:
            out_ref[pl.ds(i * _CH, _CH), :] = (2.0 * t) * y
        else:
            r = (i - half) * _CH
            out_ref[pl.ds(r, _CH), :] = (
                out_ref[pl.ds(r, _CH), :] + (2.0 - 2.0 * t) * y
            )
        if i + _NBUF < _NCHUNK:
            copy(i + _NBUF, slot).start()


@jax.jit
def kernel(adj1, mEmbed, inter):
    return pl.pallas_call(
        _ddi_gcn_kernel,
        in_specs=[
            pl.BlockSpec(memory_space=pltpu.HBM),
            pl.BlockSpec(memory_space=pltpu.VMEM),
            pl.BlockSpec(memory_space=pltpu.VMEM),
        ],
        out_specs=pl.BlockSpec(memory_space=pltpu.VMEM),
        out_shape=jax.ShapeDtypeStruct((_MEDNUM, _FDIM), jnp.float32),
        scratch_shapes=[
            pltpu.VMEM((_NBUF, _CH, 2 * _MEDNUM), jnp.float32),
            pltpu.SemaphoreType.DMA((_NBUF,)),
        ],
    )(adj1, mEmbed, inter.reshape(1, 1))


# speedup vs baseline: 1.0756x; 1.0756x over previous
import jax
import jax.numpy as jnp
from jax.experimental import pallas as pl
from jax.experimental.pallas import tpu as pltpu

_MEDNUM = 2048
_FDIM = 64
_BR = 256


def _ddi_gcn_kernel(adj_ref, w_ref, inter_ref, out_ref):
    w = w_ref[:].astype(jnp.bfloat16)
    a1 = (adj_ref[0, :, :_MEDNUM] + adj_ref[0, :, _MEDNUM:]).astype(jnp.bfloat16)
    a2 = (adj_ref[1, :, :_MEDNUM] + adj_ref[1, :, _MEDNUM:]).astype(jnp.bfloat16)
    y1 = jnp.maximum(jnp.dot(a1, w, preferred_element_type=jnp.float32), 0.0)
    y2 = jnp.maximum(jnp.dot(a2, w, preferred_element_type=jnp.float32), 0.0)
    t = inter_ref[0, 0]
    out_ref[:] = (2.0 * t) * y1 + (2.0 - 2.0 * t) * y2


@jax.jit
def kernel(adj1, mEmbed, inter):
    n_tiles = _MEDNUM // _BR
    adj3 = adj1.reshape(2, _MEDNUM, 2 * _MEDNUM)
    return pl.pallas_call(
        _ddi_gcn_kernel,
        grid=(n_tiles,),
        in_specs=[
            pl.BlockSpec((2, _BR, 2 * _MEDNUM), lambda j: (0, j, 0)),
            pl.BlockSpec((_MEDNUM, _FDIM), lambda j: (0, 0)),
            pl.BlockSpec((1, 1), lambda j: (0, 0)),
        ],
        out_specs=pl.BlockSpec((_BR, _FDIM), lambda j: (j, 0)),
        out_shape=jax.ShapeDtypeStruct((_MEDNUM, _FDIM), jnp.float32),
    )(adj3, mEmbed, inter.reshape(1, 1))
